# Initial kernel scaffold; baseline (speedup 1.0000x reference)
import jax
import jax.numpy as jnp
from jax.experimental import pallas as pl


def kernel(x, edge_index, W1, W2, W3, eps):
    raise NotImplementedError("write your pallas kernel here")


# speedup vs baseline: 7.1506x; 7.1506x over previous
import functools

import jax
import jax.numpy as jnp
from jax import lax
from jax.experimental import pallas as pl
from jax.experimental.pallas import tpu as pltpu
from jax.experimental.pallas import tpu_sc as plsc

N = 10000
D = 128
H1 = 64
H2 = 32

NC = 2
NS = 16
NW = NC * NS
CH = 128
NCH = 79
E_PAD = NW * NCH * CH
N_PAD = 10112
RPT = N_PAD // NS


def _sc_body(table, src_idx, dst_idx, zeros, out, idx_s, idx_d, rows, acc, sem):
    c = lax.axis_index("c")
    s = lax.axis_index("s")
    wid = c * NS + s
    r0 = s * RPT

    pltpu.sync_copy(zeros.at[pl.ds(r0, RPT)], acc.at[pl.ds(r0, RPT)])
    plsc.subcore_barrier()

    def chunk(j, carry):
        row = wid * NCH + j
        pltpu.sync_copy(src_idx.at[row], idx_s)
        pltpu.sync_copy(dst_idx.at[row], idx_d)
        pltpu.async_copy(table.at[idx_s], rows, sem).wait()
        pltpu.sync_copy(rows, acc.at[idx_d], add=True)
        return carry

    lax.fori_loop(0, NCH, chunk, 0)
    plsc.subcore_barrier()
    pltpu.sync_copy(acc.at[pl.ds(r0, RPT)], out.at[c, pl.ds(r0, RPT)])


_sc_scatter = functools.partial(
    pl.kernel,
    out_type=jax.ShapeDtypeStruct((NC, N_PAD, H1), jnp.float32),
    mesh=plsc.VectorSubcoreMesh(core_axis_name="c", subcore_axis_name="s"),
    scratch_types=[
        pltpu.VMEM((CH,), jnp.int32),
        pltpu.VMEM((CH,), jnp.int32),
        pltpu.VMEM((CH, H1), jnp.float32),
        pltpu.VMEM_SHARED((N_PAD, H1), jnp.float32),
        pltpu.SemaphoreType.DMA,
    ],
    compiler_params=pltpu.CompilerParams(use_tc_tiling_on_sc=False),
)(_sc_body)


BM = 400


def _mm_body(x_ref, w_ref, o_ref):
    o_ref[...] = jnp.dot(x_ref[...], w_ref[...],
                         preferred_element_type=jnp.float32)


def _first_matmul(x, w1):
    return pl.pallas_call(
        _mm_body,
        grid=(N // BM,),
        in_specs=[
            pl.BlockSpec((BM, D), lambda i: (i, 0)),
            pl.BlockSpec((D, H1), lambda i: (0, 0)),
        ],
        out_specs=pl.BlockSpec((BM, H1), lambda i: (i, 0)),
        out_shape=jax.ShapeDtypeStruct((N, H1), jnp.float32),
    )(x, w1)


def _mid_body(p_ref, w_ref, o_ref):
    h = p_ref[0] + p_ref[1]
    o_ref[...] = jnp.dot(h, w_ref[...], preferred_element_type=jnp.float32)


def _mid_matmul(partials, w23):
    return pl.pallas_call(
        _mid_body,
        grid=(N // BM,),
        in_specs=[
            pl.BlockSpec((NC, BM, H1), lambda i: (0, i, 0)),
            pl.BlockSpec((H1, H1), lambda i: (0, 0)),
        ],
        out_specs=pl.BlockSpec((BM, H1), lambda i: (i, 0)),
        out_shape=jax.ShapeDtypeStruct((N, H1), jnp.float32),
    )(partials, w23)


def _final_body(p_ref, eps_ref, z_ref, mu_ref, lv_ref):
    g = p_ref[0] + p_ref[1]
    mu = g[:, :H2]
    lv = g[:, H2:]
    mu_ref[...] = mu
    lv_ref[...] = lv
    z_ref[...] = eps_ref[...] * jnp.exp(lv) + mu


def _final_stage(partials, eps):
    shp = jax.ShapeDtypeStruct((N, H2), jnp.float32)
    return pl.pallas_call(
        _final_body,
        grid=(N // BM,),
        in_specs=[
            pl.BlockSpec((NC, BM, H1), lambda i: (0, i, 0)),
            pl.BlockSpec((BM, H2), lambda i: (i, 0)),
        ],
        out_specs=[
            pl.BlockSpec((BM, H2), lambda i: (i, 0)),
            pl.BlockSpec((BM, H2), lambda i: (i, 0)),
            pl.BlockSpec((BM, H2), lambda i: (i, 0)),
        ],
        out_shape=[shp, shp, shp],
    )(partials, eps)


def kernel(x, edge_index, W1, W2, W3, eps):
    pad = E_PAD - edge_index.shape[1]
    src = jnp.concatenate(
        [edge_index[0], jnp.zeros((pad,), jnp.int32)]).reshape(NW * NCH, CH)
    dst = jnp.concatenate(
        [edge_index[1], jnp.full((pad,), N, jnp.int32)]).reshape(NW * NCH, CH)
    zeros = jnp.zeros((N_PAD, H1), jnp.float32)
    w23 = jnp.concatenate([W2, W3], axis=1)

    s1 = _first_matmul(x, W1)
    p1 = _sc_scatter(s1, src, dst, zeros)
    s2 = _mid_matmul(p1, w23)
    p2 = _sc_scatter(s2, src, dst, zeros)
    z, mu, logvar = _final_stage(p2, eps)
    return (z, mu, logvar)


# speedup vs baseline: 10.8266x; 1.5141x over previous
import functools

import jax
import jax.numpy as jnp
from jax import lax
from jax.experimental import pallas as pl
from jax.experimental.pallas import tpu as pltpu
from jax.experimental.pallas import tpu_sc as plsc

N = 10000
D = 128
H1 = 64
H2 = 32

NC = 2
NS = 16
NW = NC * NS
CH = 128
NCH = 79
E_PAD = NW * NCH * CH
N_PAD = 10112
RPT = N_PAD // NS


def _sc_body(table, src_idx, dst_idx, zeros, out,
             idx_s, idx_d, rows, acc, gsem, ssem):
    c = lax.axis_index("c")
    s = lax.axis_index("s")
    wid = c * NS + s
    r0 = s * RPT

    pltpu.sync_copy(zeros.at[pl.ds(r0, RPT)], acc.at[pl.ds(r0, RPT)])
    pltpu.sync_copy(src_idx.at[pl.ds(wid * NCH, NCH)], idx_s)
    pltpu.sync_copy(dst_idx.at[pl.ds(wid * NCH, NCH)], idx_d)
    plsc.subcore_barrier()

    pltpu.async_copy(table.at[idx_s.at[0]], rows.at[0], gsem)

    def chunk(j, carry):
        b = j & 1

        @pl.when(j >= 1)
        def _drain_prev_scatter():
            pltpu.make_async_copy(
                rows.at[1 - b], acc.at[idx_d.at[j - 1]], ssem).wait()

        @pl.when(j + 1 < NCH)
        def _fire_next_gather():
            pltpu.async_copy(table.at[idx_s.at[j + 1]], rows.at[1 - b], gsem)

        pltpu.make_async_copy(table.at[idx_s.at[j]], rows.at[b], gsem).wait()
        pltpu.async_copy(rows.at[b], acc.at[idx_d.at[j]], ssem, add=True)
        return carry

    lax.fori_loop(0, NCH, chunk, 0)
    pltpu.make_async_copy(
        rows.at[(NCH - 1) & 1], acc.at[idx_d.at[NCH - 1]], ssem).wait()
    plsc.subcore_barrier()
    pltpu.sync_copy(acc.at[pl.ds(r0, RPT)], out.at[c, pl.ds(r0, RPT)])


_sc_scatter = functools.partial(
    pl.kernel,
    out_type=jax.ShapeDtypeStruct((NC, N_PAD, H1), jnp.float32),
    mesh=plsc.VectorSubcoreMesh(core_axis_name="c", subcore_axis_name="s"),
    scratch_types=[
        pltpu.VMEM((NCH, CH), jnp.int32),
        pltpu.VMEM((NCH, CH), jnp.int32),
        pltpu.VMEM((2, CH, H1), jnp.float32),
        pltpu.VMEM_SHARED((N_PAD, H1), jnp.float32),
        pltpu.SemaphoreType.DMA,
        pltpu.SemaphoreType.DMA,
    ],
    compiler_params=pltpu.CompilerParams(use_tc_tiling_on_sc=False),
)(_sc_body)


BM = 400


def _mm_body(x_ref, w_ref, o_ref):
    o_ref[...] = jnp.dot(x_ref[...], w_ref[...],
                         preferred_element_type=jnp.float32)


def _first_matmul(x, w1):
    return pl.pallas_call(
        _mm_body,
        grid=(N // BM,),
        in_specs=[
            pl.BlockSpec((BM, D), lambda i: (i, 0)),
            pl.BlockSpec((D, H1), lambda i: (0, 0)),
        ],
        out_specs=pl.BlockSpec((BM, H1), lambda i: (i, 0)),
        out_shape=jax.ShapeDtypeStruct((N, H1), jnp.float32),
    )(x, w1)


def _mid_body(p_ref, w_ref, o_ref):
    h = p_ref[0] + p_ref[1]
    o_ref[...] = jnp.dot(h, w_ref[...], preferred_element_type=jnp.float32)


def _mid_matmul(partials, w23):
    return pl.pallas_call(
        _mid_body,
        grid=(N // BM,),
        in_specs=[
            pl.BlockSpec((NC, BM, H1), lambda i: (0, i, 0)),
            pl.BlockSpec((H1, H1), lambda i: (0, 0)),
        ],
        out_specs=pl.BlockSpec((BM, H1), lambda i: (i, 0)),
        out_shape=jax.ShapeDtypeStruct((N, H1), jnp.float32),
    )(partials, w23)


def _final_body(p_ref, eps_ref, z_ref, mu_ref, lv_ref):
    g = p_ref[0] + p_ref[1]
    mu = g[:, :H2]
    lv = g[:, H2:]
    mu_ref[...] = mu
    lv_ref[...] = lv
    z_ref[...] = eps_ref[...] * jnp.exp(lv) + mu


def _final_stage(partials, eps):
    shp = jax.ShapeDtypeStruct((N, H2), jnp.float32)
    return pl.pallas_call(
        _final_body,
        grid=(N // BM,),
        in_specs=[
            pl.BlockSpec((NC, BM, H1), lambda i: (0, i, 0)),
            pl.BlockSpec((BM, H2), lambda i: (i, 0)),
        ],
        out_specs=[
            pl.BlockSpec((BM, H2), lambda i: (i, 0)),
            pl.BlockSpec((BM, H2), lambda i: (i, 0)),
            pl.BlockSpec((BM, H2), lambda i: (i, 0)),
        ],
        out_shape=[shp, shp, shp],
    )(partials, eps)


def kernel(x, edge_index, W1, W2, W3, eps):
    pad = E_PAD - edge_index.shape[1]
    src = jnp.concatenate(
        [edge_index[0], jnp.zeros((pad,), jnp.int32)]).reshape(NW * NCH, CH)
    dst = jnp.concatenate(
        [edge_index[1], jnp.full((pad,), N, jnp.int32)]).reshape(NW * NCH, CH)
    zeros = jnp.zeros((N_PAD, H1), jnp.float32)
    w23 = jnp.concatenate([W2, W3], axis=1)

    s1 = _first_matmul(x, W1)
    p1 = _sc_scatter(s1, src, dst, zeros)
    s2 = _mid_matmul(p1, w23)
    p2 = _sc_scatter(s2, src, dst, zeros)
    z, mu, logvar = _final_stage(p2, eps)
    return (z, mu, logvar)


# speedup vs baseline: 11.2592x; 1.0400x over previous
import functools

import jax
import jax.numpy as jnp
from jax import lax
from jax.experimental import pallas as pl
from jax.experimental.pallas import tpu as pltpu
from jax.experimental.pallas import tpu_sc as plsc

N = 10000
D = 128
H1 = 64
H2 = 32

NC = 2
NS = 16
NW = NC * NS
CH = 128
NCH = 79
E_PAD = NW * NCH * CH
N_PAD = 10112
RPT = N_PAD // NS
NBUF = 8
LAG = NBUF // 2


def _sc_body(table, src_idx, dst_idx, zeros, out,
             idx_s, idx_d, rows, acc, gsem, ssem):
    c = lax.axis_index("c")
    s = lax.axis_index("s")
    wid = c * NS + s
    r0 = s * RPT

    pltpu.sync_copy(zeros.at[pl.ds(r0, RPT)], acc.at[pl.ds(r0, RPT)])
    pltpu.sync_copy(src_idx.at[pl.ds(wid * NCH, NCH)], idx_s)
    pltpu.sync_copy(dst_idx.at[pl.ds(wid * NCH, NCH)], idx_d)
    plsc.subcore_barrier()

    for t in range(LAG):
        pltpu.async_copy(table.at[idx_s.at[t]], rows.at[t], gsem)

    def chunk(j, carry):
        b = lax.rem(j, NBUF)
        bnext = lax.rem(j + LAG, NBUF)

        @pl.when(j >= LAG)
        def _drain_lagged_scatter():
            pltpu.make_async_copy(
                rows.at[bnext], acc.at[idx_d.at[j - LAG]], ssem).wait()

        @pl.when(j + LAG < NCH)
        def _fire_ahead_gather():
            pltpu.async_copy(table.at[idx_s.at[j + LAG]], rows.at[bnext], gsem)

        pltpu.make_async_copy(table.at[idx_s.at[j]], rows.at[b], gsem).wait()
        pltpu.async_copy(rows.at[b], acc.at[idx_d.at[j]], ssem, add=True)
        return carry

    lax.fori_loop(0, NCH, chunk, 0)
    for t in range(max(NCH - LAG, 0), NCH):
        pltpu.make_async_copy(
            rows.at[t % NBUF], acc.at[idx_d.at[t]], ssem).wait()
    plsc.subcore_barrier()
    pltpu.sync_copy(acc.at[pl.ds(r0, RPT)], out.at[c, pl.ds(r0, RPT)])


_sc_scatter = functools.partial(
    pl.kernel,
    out_type=jax.ShapeDtypeStruct((NC, N_PAD, H1), jnp.float32),
    mesh=plsc.VectorSubcoreMesh(core_axis_name="c", subcore_axis_name="s"),
    scratch_types=[
        pltpu.VMEM((NCH, CH), jnp.int32),
        pltpu.VMEM((NCH, CH), jnp.int32),
        pltpu.VMEM((NBUF, CH, H1), jnp.float32),
        pltpu.VMEM_SHARED((N_PAD, H1), jnp.float32),
        pltpu.SemaphoreType.DMA,
        pltpu.SemaphoreType.DMA,
    ],
    compiler_params=pltpu.CompilerParams(use_tc_tiling_on_sc=False),
)(_sc_body)


BM = 400


def _mm_body(x_ref, w_ref, o_ref):
    o_ref[...] = jnp.dot(x_ref[...], w_ref[...],
                         preferred_element_type=jnp.float32)


def _first_matmul(x, w1):
    return pl.pallas_call(
        _mm_body,
        grid=(N // BM,),
        in_specs=[
            pl.BlockSpec((BM, D), lambda i: (i, 0)),
            pl.BlockSpec((D, H1), lambda i: (0, 0)),
        ],
        out_specs=pl.BlockSpec((BM, H1), lambda i: (i, 0)),
        out_shape=jax.ShapeDtypeStruct((N, H1), jnp.float32),
    )(x, w1)


def _mid_body(p_ref, w_ref, o_ref):
    h = p_ref[0] + p_ref[1]
    o_ref[...] = jnp.dot(h, w_ref[...], preferred_element_type=jnp.float32)


def _mid_matmul(partials, w23):
    return pl.pallas_call(
        _mid_body,
        grid=(N // BM,),
        in_specs=[
            pl.BlockSpec((NC, BM, H1), lambda i: (0, i, 0)),
            pl.BlockSpec((H1, H1), lambda i: (0, 0)),
        ],
        out_specs=pl.BlockSpec((BM, H1), lambda i: (i, 0)),
        out_shape=jax.ShapeDtypeStruct((N, H1), jnp.float32),
    )(partials, w23)


def _final_body(p_ref, eps_ref, z_ref, mu_ref, lv_ref):
    g = p_ref[0] + p_ref[1]
    mu = g[:, :H2]
    lv = g[:, H2:]
    mu_ref[...] = mu
    lv_ref[...] = lv
    z_ref[...] = eps_ref[...] * jnp.exp(lv) + mu


def _final_stage(partials, eps):
    shp = jax.ShapeDtypeStruct((N, H2), jnp.float32)
    return pl.pallas_call(
        _final_body,
        grid=(N // BM,),
        in_specs=[
            pl.BlockSpec((NC, BM, H1), lambda i: (0, i, 0)),
            pl.BlockSpec((BM, H2), lambda i: (i, 0)),
        ],
        out_specs=[
            pl.BlockSpec((BM, H2), lambda i: (i, 0)),
            pl.BlockSpec((BM, H2), lambda i: (i, 0)),
            pl.BlockSpec((BM, H2), lambda i: (i, 0)),
        ],
        out_shape=[shp, shp, shp],
    )(partials, eps)


def kernel(x, edge_index, W1, W2, W3, eps):
    pad = E_PAD - edge_index.shape[1]
    src = jnp.concatenate(
        [edge_index[0], jnp.zeros((pad,), jnp.int32)]).reshape(NW * NCH, CH)
    dst = jnp.concatenate(
        [edge_index[1], jnp.full((pad,), N, jnp.int32)]).reshape(NW * NCH, CH)
    zeros = jnp.zeros((N_PAD, H1), jnp.float32)
    w23 = jnp.concatenate([W2, W3], axis=1)

    s1 = _first_matmul(x, W1)
    p1 = _sc_scatter(s1, src, dst, zeros)
    s2 = _mid_matmul(p1, w23)
    p2 = _sc_scatter(s2, src, dst, zeros)
    z, mu, logvar = _final_stage(p2, eps)
    return (z, mu, logvar)


# speedup vs baseline: 15.8213x; 1.4052x over previous
import functools

import jax
import jax.numpy as jnp
from jax import lax
from jax.experimental import pallas as pl
from jax.experimental.pallas import tpu as pltpu
from jax.experimental.pallas import tpu_sc as plsc

N = 10000
D = 128
H1 = 64
H2 = 32

NC = 2
NS = 16
NW = NC * NS
CH = 128
NCH = 79
E_PAD = NW * NCH * CH
N_PAD = 10112
RPT = N_PAD // NS
NBUF = 2
LAG = NBUF // 2


def _sc_body(table, src_idx, dst_idx, zeros, out,
             idx_s, idx_d, rows, acc, tbl, gsem, ssem):
    c = lax.axis_index("c")
    s = lax.axis_index("s")
    wid = c * NS + s
    r0 = s * RPT

    pltpu.sync_copy(zeros.at[pl.ds(r0, RPT)], acc.at[pl.ds(r0, RPT)])
    @pl.when(s < NS - 1)
    def _stage_main():
        pltpu.sync_copy(table.at[pl.ds(s * RPT, RPT)],
                        tbl.at[pl.ds(s * RPT, RPT)])

    @pl.when(s == NS - 1)
    def _stage_tail():
        pltpu.sync_copy(table.at[pl.ds((NS - 1) * RPT, N - (NS - 1) * RPT)],
                        tbl.at[pl.ds((NS - 1) * RPT, N - (NS - 1) * RPT)])

    pltpu.sync_copy(src_idx.at[pl.ds(wid * NCH, NCH)], idx_s)
    pltpu.sync_copy(dst_idx.at[pl.ds(wid * NCH, NCH)], idx_d)
    plsc.subcore_barrier()

    for t in range(LAG):
        pltpu.async_copy(tbl.at[idx_s.at[t]], rows.at[t], gsem)

    def chunk(j, carry):
        b = lax.rem(j, NBUF)
        bnext = lax.rem(j + LAG, NBUF)

        @pl.when(j >= LAG)
        def _drain_lagged_scatter():
            pltpu.make_async_copy(
                rows.at[bnext], acc.at[idx_d.at[j - LAG]], ssem).wait()

        @pl.when(j + LAG < NCH)
        def _fire_ahead_gather():
            pltpu.async_copy(tbl.at[idx_s.at[j + LAG]], rows.at[bnext], gsem)

        pltpu.make_async_copy(tbl.at[idx_s.at[j]], rows.at[b], gsem).wait()
        pltpu.async_copy(rows.at[b], acc.at[idx_d.at[j]], ssem, add=True)
        return carry

    lax.fori_loop(0, NCH, chunk, 0)
    for t in range(max(NCH - LAG, 0), NCH):
        pltpu.make_async_copy(
            rows.at[t % NBUF], acc.at[idx_d.at[t]], ssem).wait()
    plsc.subcore_barrier()
    pltpu.sync_copy(acc.at[pl.ds(r0, RPT)], out.at[c, pl.ds(r0, RPT)])


_sc_scatter = functools.partial(
    pl.kernel,
    out_type=jax.ShapeDtypeStruct((NC, N_PAD, H1), jnp.float32),
    mesh=plsc.VectorSubcoreMesh(core_axis_name="c", subcore_axis_name="s"),
    scratch_types=[
        pltpu.VMEM((NCH, CH), jnp.int32),
        pltpu.VMEM((NCH, CH), jnp.int32),
        pltpu.VMEM((NBUF, CH, H1), jnp.float32),
        pltpu.VMEM_SHARED((N_PAD, H1), jnp.float32),
        pltpu.VMEM_SHARED((N, H1), jnp.float32),
        pltpu.SemaphoreType.DMA,
        pltpu.SemaphoreType.DMA,
    ],
    compiler_params=pltpu.CompilerParams(use_tc_tiling_on_sc=False),
)(_sc_body)


BM = 400


def _mm_body(x_ref, w_ref, o_ref):
    o_ref[...] = jnp.dot(x_ref[...], w_ref[...],
                         preferred_element_type=jnp.float32)


def _first_matmul(x, w1):
    return pl.pallas_call(
        _mm_body,
        grid=(N // BM,),
        in_specs=[
            pl.BlockSpec((BM, D), lambda i: (i, 0)),
            pl.BlockSpec((D, H1), lambda i: (0, 0)),
        ],
        out_specs=pl.BlockSpec((BM, H1), lambda i: (i, 0)),
        out_shape=jax.ShapeDtypeStruct((N, H1), jnp.float32),
    )(x, w1)


def _mid_body(p_ref, w_ref, o_ref):
    h = p_ref[0] + p_ref[1]
    o_ref[...] = jnp.dot(h, w_ref[...], preferred_element_type=jnp.float32)


def _mid_matmul(partials, w23):
    return pl.pallas_call(
        _mid_body,
        grid=(N // BM,),
        in_specs=[
            pl.BlockSpec((NC, BM, H1), lambda i: (0, i, 0)),
            pl.BlockSpec((H1, H1), lambda i: (0, 0)),
        ],
        out_specs=pl.BlockSpec((BM, H1), lambda i: (i, 0)),
        out_shape=jax.ShapeDtypeStruct((N, H1), jnp.float32),
    )(partials, w23)


def _final_body(p_ref, eps_ref, z_ref, mu_ref, lv_ref):
    g = p_ref[0] + p_ref[1]
    mu = g[:, :H2]
    lv = g[:, H2:]
    mu_ref[...] = mu
    lv_ref[...] = lv
    z_ref[...] = eps_ref[...] * jnp.exp(lv) + mu


def _final_stage(partials, eps):
    shp = jax.ShapeDtypeStruct((N, H2), jnp.float32)
    return pl.pallas_call(
        _final_body,
        grid=(N // BM,),
        in_specs=[
            pl.BlockSpec((NC, BM, H1), lambda i: (0, i, 0)),
            pl.BlockSpec((BM, H2), lambda i: (i, 0)),
        ],
        out_specs=[
            pl.BlockSpec((BM, H2), lambda i: (i, 0)),
            pl.BlockSpec((BM, H2), lambda i: (i, 0)),
            pl.BlockSpec((BM, H2), lambda i: (i, 0)),
        ],
        out_shape=[shp, shp, shp],
    )(partials, eps)


def kernel(x, edge_index, W1, W2, W3, eps):
    pad = E_PAD - edge_index.shape[1]
    src = jnp.concatenate(
        [edge_index[0], jnp.zeros((pad,), jnp.int32)]).reshape(NW * NCH, CH)
    dst = jnp.concatenate(
        [edge_index[1], jnp.full((pad,), N, jnp.int32)]).reshape(NW * NCH, CH)
    zeros = jnp.zeros((N_PAD, H1), jnp.float32)
    w23 = jnp.concatenate([W2, W3], axis=1)

    s1 = _first_matmul(x, W1)
    p1 = _sc_scatter(s1, src, dst, zeros)
    s2 = _mid_matmul(p1, w23)
    p2 = _sc_scatter(s2, src, dst, zeros)
    z, mu, logvar = _final_stage(p2, eps)
    return (z, mu, logvar)


# speedup vs baseline: 21.4412x; 1.3552x over previous
import functools

import jax
import jax.numpy as jnp
from jax import lax
from jax.experimental import pallas as pl
from jax.experimental.pallas import tpu as pltpu
from jax.experimental.pallas import tpu_sc as plsc

N = 10000
D = 128
H1 = 64
H2 = 32
HW = 32

NC = 2
NS = 16
CH = 128
NCH = 160
E_PAD = NS * NCH * CH
N_PAD = 10112
RPT = N_PAD // NS
TAIL = N - (NS - 1) * RPT
NBUF = 8
LAG = NBUF // 2


def _sc_body(table, src_idx, dst_idx, zeros, out,
             idx_s, idx_d, rows, acc, tbl, gsem, ssem):
    c = lax.axis_index("c")
    s = lax.axis_index("s")
    r0 = s * RPT
    col0 = c * HW

    pltpu.sync_copy(zeros.at[pl.ds(r0, RPT)], acc.at[pl.ds(r0, RPT)])
    @pl.when(s < NS - 1)
    def _stage_main():
        pltpu.sync_copy(table.at[pl.ds(r0, RPT), pl.ds(col0, HW)],
                        tbl.at[pl.ds(r0, RPT)])

    @pl.when(s == NS - 1)
    def _stage_tail():
        pltpu.sync_copy(table.at[pl.ds((NS - 1) * RPT, TAIL), pl.ds(col0, HW)],
                        tbl.at[pl.ds((NS - 1) * RPT, TAIL)])

    pltpu.sync_copy(src_idx.at[pl.ds(s * NCH, NCH)], idx_s)
    pltpu.sync_copy(dst_idx.at[pl.ds(s * NCH, NCH)], idx_d)
    plsc.subcore_barrier()

    for t in range(LAG):
        pltpu.async_copy(tbl.at[idx_s.at[t]], rows.at[t], gsem)

    def chunk(j, carry):
        b = lax.rem(j, NBUF)
        bnext = lax.rem(j + LAG, NBUF)

        @pl.when(j >= LAG)
        def _drain_lagged_scatter():
            pltpu.make_async_copy(
                rows.at[bnext], acc.at[idx_d.at[j - LAG]], ssem).wait()

        @pl.when(j + LAG < NCH)
        def _fire_ahead_gather():
            pltpu.async_copy(tbl.at[idx_s.at[j + LAG]], rows.at[bnext], gsem)

        pltpu.make_async_copy(tbl.at[idx_s.at[j]], rows.at[b], gsem).wait()
        pltpu.async_copy(rows.at[b], acc.at[idx_d.at[j]], ssem, add=True)
        return carry

    lax.fori_loop(0, NCH, chunk, 0)
    for t in range(max(NCH - LAG, 0), NCH):
        pltpu.make_async_copy(
            rows.at[t % NBUF], acc.at[idx_d.at[t]], ssem).wait()
    plsc.subcore_barrier()
    pltpu.sync_copy(acc.at[pl.ds(r0, RPT)],
                    out.at[pl.ds(r0, RPT), pl.ds(col0, HW)])


_sc_scatter = functools.partial(
    pl.kernel,
    out_type=jax.ShapeDtypeStruct((N_PAD, H1), jnp.float32),
    mesh=plsc.VectorSubcoreMesh(core_axis_name="c", subcore_axis_name="s"),
    scratch_types=[
        pltpu.VMEM((NCH, CH), jnp.int32),
        pltpu.VMEM((NCH, CH), jnp.int32),
        pltpu.VMEM((NBUF, CH, HW), jnp.float32),
        pltpu.VMEM_SHARED((N_PAD, HW), jnp.float32),
        pltpu.VMEM_SHARED((N, HW), jnp.float32),
        pltpu.SemaphoreType.DMA,
        pltpu.SemaphoreType.DMA,
    ],
    compiler_params=pltpu.CompilerParams(use_tc_tiling_on_sc=False),
)(_sc_body)


BM = 2000


def _mm_body(x_ref, w_ref, o_ref):
    o_ref[...] = jnp.dot(x_ref[...], w_ref[...],
                         preferred_element_type=jnp.float32)


def _first_matmul(x, w1):
    return pl.pallas_call(
        _mm_body,
        grid=(N // BM,),
        in_specs=[
            pl.BlockSpec((BM, D), lambda i: (i, 0)),
            pl.BlockSpec((D, H1), lambda i: (0, 0)),
        ],
        out_specs=pl.BlockSpec((BM, H1), lambda i: (i, 0)),
        out_shape=jax.ShapeDtypeStruct((N_PAD, H1), jnp.float32),
    )(x, w1)


def _final_body(g_ref, eps_ref, w2_ref, w3_ref, z_ref, mu_ref, lv_ref):
    g = g_ref[...]
    mu = jnp.dot(g, w2_ref[...], preferred_element_type=jnp.float32)
    lv = jnp.dot(g, w3_ref[...], preferred_element_type=jnp.float32)
    mu_ref[...] = mu
    lv_ref[...] = lv
    z_ref[...] = eps_ref[...] * jnp.exp(lv) + mu


def _final_stage(g, eps, w2, w3):
    shp = jax.ShapeDtypeStruct((N, H2), jnp.float32)
    return pl.pallas_call(
        _final_body,
        grid=(N // BM,),
        in_specs=[
            pl.BlockSpec((BM, H1), lambda i: (i, 0)),
            pl.BlockSpec((BM, H2), lambda i: (i, 0)),
            pl.BlockSpec((H1, H2), lambda i: (0, 0)),
            pl.BlockSpec((H1, H2), lambda i: (0, 0)),
        ],
        out_specs=[
            pl.BlockSpec((BM, H2), lambda i: (i, 0)),
            pl.BlockSpec((BM, H2), lambda i: (i, 0)),
            pl.BlockSpec((BM, H2), lambda i: (i, 0)),
        ],
        out_shape=[shp, shp, shp],
    )(g, eps, w2, w3)


def kernel(x, edge_index, W1, W2, W3, eps):
    pad = E_PAD - edge_index.shape[1]
    src = jnp.concatenate(
        [edge_index[0], jnp.zeros((pad,), jnp.int32)]).reshape(NS * NCH, CH)
    dst = jnp.concatenate(
        [edge_index[1], jnp.full((pad,), N, jnp.int32)]).reshape(NS * NCH, CH)
    zeros = jnp.zeros((N_PAD, HW), jnp.float32)

    s1 = _first_matmul(x, W1)
    h1 = _sc_scatter(s1, src, dst, zeros)
    g = _sc_scatter(h1, src, dst, zeros)
    z, mu, logvar = _final_stage(g, eps, W2, W3)
    return (z, mu, logvar)
